# baseline (speedup 1.0000x reference)
import functools

import jax
import jax.numpy as jnp
from jax import lax
from jax.experimental import pallas as pl
from jax.experimental.pallas import tpu as pltpu
from jax.experimental.pallas import tpu_sc as plsc

_BANK = 1000001
_DIM = 64
_BATCH = 16384
_MOM = 0.9

_NC, _NS = 2, 16
_NW = _NC * _NS
_BPW = _BATCH // _NW
_CH = 128
_NCH = _BPW // _CH

_N_DMA = 8
_CHUNK = _BANK // _N_DMA
_COPY_SLABS = [(i * _CHUNK, _CHUNK) for i in range(_N_DMA - 1)]
_COPY_SLABS.append(((_N_DMA - 1) * _CHUNK, _BANK - (_N_DMA - 1) * _CHUNK))


def _copy_body(m_ref, o_ref, sem):
    for start, size in _COPY_SLABS:
        pltpu.async_copy(m_ref.at[pl.ds(start, size)],
                         o_ref.at[pl.ds(start, size)], sem)
    for start, size in _COPY_SLABS:
        pltpu.make_async_copy(m_ref.at[pl.ds(start, size)],
                              o_ref.at[pl.ds(start, size)], sem).wait()


def _bank_copy(memory):
    return pl.pallas_call(
        _copy_body,
        in_specs=[pl.BlockSpec(memory_space=pltpu.HBM)],
        out_specs=pl.BlockSpec(memory_space=pltpu.HBM),
        out_shape=jax.ShapeDtypeStruct((_BANK, _DIM), jnp.float32),
        scratch_shapes=[pltpu.SemaphoreType.DMA],
    )(memory)


@functools.partial(
    pl.kernel,
    out_type=jax.ShapeDtypeStruct((_BATCH, _DIM), jnp.float32),
    mesh=plsc.VectorSubcoreMesh(core_axis_name="c", subcore_axis_name="s"),
    compiler_params=pltpu.CompilerParams(use_tc_tiling_on_sc=False),
    scratch_types=[
        pltpu.VMEM((_NCH, _CH), jnp.int32),
        pltpu.VMEM((_BPW, _DIM), jnp.float32),
        pltpu.VMEM((_BPW, _DIM), jnp.float32),
        pltpu.SemaphoreType.DMA,
    ],
)
def _sc_update(idx_hbm, data_hbm, mem_hbm, upd_ref, avgs_hbm,
               idx_v, rows_v, data_v, sem):
    wid = lax.axis_index("s") * _NC + lax.axis_index("c")
    base = wid * _BPW

    pltpu.sync_copy(idx_hbm.at[pl.ds(wid * _NCH, _NCH)], idx_v)

    for j in range(_NCH):
        pltpu.async_copy(mem_hbm.at[idx_v.at[j]],
                         rows_v.at[pl.ds(j * _CH, _CH)], sem)
    for j in range(_NCH):
        pltpu.make_async_copy(mem_hbm.at[idx_v.at[j]],
                              rows_v.at[pl.ds(j * _CH, _CH)], sem).wait()

    pltpu.sync_copy(rows_v, avgs_hbm.at[pl.ds(base, _BPW)])
    pltpu.sync_copy(data_hbm.at[pl.ds(base, _BPW)], data_v)

    def body(i, carry):
        for k in range(_DIM // 16):
            sl = pl.ds(k * 16, 16)
            rows_v[i, sl] = rows_v[i, sl] * _MOM + data_v[i, sl] * (1.0 - _MOM)
        return carry

    lax.fori_loop(0, _BPW, body, 0)

    for j in range(_NCH):
        pltpu.async_copy(rows_v.at[pl.ds(j * _CH, _CH)],
                         upd_ref.at[idx_v.at[j]], sem)
    for j in range(_NCH):
        pltpu.make_async_copy(rows_v.at[pl.ds(j * _CH, _CH)],
                              upd_ref.at[idx_v.at[j]], sem).wait()


def kernel(idx, data, memory):
    idx2d = idx.astype(jnp.int32).reshape(_NW * _NCH, _CH)
    bank = _bank_copy(memory)
    bank_ref = jax.new_ref(bank)
    avgs = _sc_update(idx2d, data, memory, bank_ref)
    return avgs, bank_ref[...]


# speedup vs baseline: 12.1929x; 12.1929x over previous
import functools

import jax
import jax.numpy as jnp
from jax import lax
from jax.experimental import pallas as pl
from jax.experimental.pallas import tpu as pltpu
from jax.experimental.pallas import tpu_sc as plsc

_BANK = 1000001
_DIM = 64
_BATCH = 16384
_MOM = 0.9

_NC, _NS = 2, 16
_NW = _NC * _NS
_BPW = _BATCH // _NW
_CH = 128
_NCH = _BPW // _CH

_CPW = 31248
_CROWS = 512
_NFULL = _CPW // _CROWS
_CTAIL = _CPW - _NFULL * _CROWS
_TAIL_BASE = _NW * _CPW

_mesh = plsc.VectorSubcoreMesh(core_axis_name="c", subcore_axis_name="s")
_sc_params = pltpu.CompilerParams(use_tc_tiling_on_sc=False)


def _wid():
    return lax.axis_index("s") * _NC + lax.axis_index("c")


@functools.partial(
    pl.kernel,
    out_type=jax.ShapeDtypeStruct((_BANK, _DIM), jnp.float32),
    mesh=_mesh,
    compiler_params=_sc_params,
    scratch_types=[
        pltpu.VMEM((2, _CROWS, _DIM), jnp.float32),
        pltpu.SemaphoreType.DMA,
        pltpu.SemaphoreType.DMA,
    ],
)
def _bank_copy(mem_hbm, out_hbm, buf, sem_i, sem_o):
    w = _wid()
    base = w * _CPW

    def in_copy(k, slot):
        return pltpu.make_async_copy(
            mem_hbm.at[pl.ds(base + k * _CROWS, _CROWS)], buf.at[slot], sem_i)

    def out_copy(k, slot):
        return pltpu.make_async_copy(
            buf.at[slot], out_hbm.at[pl.ds(base + k * _CROWS, _CROWS)], sem_o)

    in_copy(0, 0).start()

    def body(k, carry):
        slot = lax.rem(k, 2)
        nslot = lax.rem(k + 1, 2)

        @pl.when(k + 1 < _NFULL)
        def _():
            @pl.when(k >= 1)
            def _():
                out_copy(k - 1, nslot).wait()
            in_copy(k + 1, nslot).start()

        in_copy(k, slot).wait()
        out_copy(k, slot).start()
        return carry

    lax.fori_loop(0, _NFULL, body, 0)
    out_copy(_NFULL - 2, lax.rem(_NFULL, 2)).wait()
    out_copy(_NFULL - 1, lax.rem(_NFULL - 1, 2)).wait()

    t0 = base + _NFULL * _CROWS
    pltpu.async_copy(mem_hbm.at[pl.ds(t0, _CTAIL)],
                     buf.at[0, pl.ds(0, _CTAIL)], sem_i).wait()
    pltpu.async_copy(buf.at[0, pl.ds(0, _CTAIL)],
                     out_hbm.at[pl.ds(t0, _CTAIL)], sem_o).wait()

    @pl.when(w < 8)
    def _():
        tb = _TAIL_BASE + w * 8
        pltpu.async_copy(mem_hbm.at[pl.ds(tb, 8)],
                         buf.at[1, pl.ds(0, 8)], sem_i).wait()
        pltpu.async_copy(buf.at[1, pl.ds(0, 8)],
                         out_hbm.at[pl.ds(tb, 8)], sem_o).wait()

    @pl.when(w == 8)
    def _():
        pltpu.async_copy(mem_hbm.at[pl.ds(_BANK - 1, 1)],
                         buf.at[1, pl.ds(8, 1)], sem_i).wait()
        pltpu.async_copy(buf.at[1, pl.ds(8, 1)],
                         out_hbm.at[pl.ds(_BANK - 1, 1)], sem_o).wait()


@functools.partial(
    pl.kernel,
    out_type=jax.ShapeDtypeStruct((_BATCH, _DIM), jnp.float32),
    mesh=_mesh,
    compiler_params=_sc_params,
    scratch_types=[
        pltpu.VMEM((_NCH, _CH), jnp.int32),
        pltpu.VMEM((_BPW, _DIM), jnp.float32),
        pltpu.VMEM((_BPW, _DIM), jnp.float32),
        pltpu.SemaphoreType.DMA,
    ],
)
def _sc_update(idx_hbm, data_hbm, mem_hbm, upd_ref, avgs_hbm,
               idx_v, rows_v, data_v, sem):
    w = _wid()
    base = w * _BPW

    pltpu.sync_copy(idx_hbm.at[pl.ds(w * _NCH, _NCH)], idx_v)

    for j in range(_NCH):
        pltpu.async_copy(mem_hbm.at[idx_v.at[j]],
                         rows_v.at[pl.ds(j * _CH, _CH)], sem)
    for j in range(_NCH):
        pltpu.make_async_copy(mem_hbm.at[idx_v.at[j]],
                              rows_v.at[pl.ds(j * _CH, _CH)], sem).wait()

    pltpu.sync_copy(rows_v, avgs_hbm.at[pl.ds(base, _BPW)])
    pltpu.sync_copy(data_hbm.at[pl.ds(base, _BPW)], data_v)

    def body(i, carry):
        for k in range(_DIM // 16):
            sl = pl.ds(k * 16, 16)
            rows_v[i, sl] = rows_v[i, sl] * _MOM + data_v[i, sl] * (1.0 - _MOM)
        return carry

    lax.fori_loop(0, _BPW, body, 0)

    for j in range(_NCH):
        pltpu.async_copy(rows_v.at[pl.ds(j * _CH, _CH)],
                         upd_ref.at[idx_v.at[j]], sem)
    for j in range(_NCH):
        pltpu.make_async_copy(rows_v.at[pl.ds(j * _CH, _CH)],
                              upd_ref.at[idx_v.at[j]], sem).wait()


def kernel(idx, data, memory):
    idx2d = idx.astype(jnp.int32).reshape(_NW * _NCH, _CH)
    bank = _bank_copy(memory)
    bank_ref = jax.new_ref(bank)
    avgs = _sc_update(idx2d, data, memory, bank_ref)
    return avgs, bank_ref[...]


# speedup vs baseline: 14.0116x; 1.1492x over previous
import functools

import jax
import jax.numpy as jnp
from jax import lax
from jax.experimental import pallas as pl
from jax.experimental.pallas import tpu as pltpu
from jax.experimental.pallas import tpu_sc as plsc

_BANK = 1000001
_DIM = 64
_BATCH = 16384
_MOM = 0.9

_NC, _NS = 2, 16
_NW = _NC * _NS
_BPW = _BATCH // _NW
_CH = 128
_NCH = _BPW // _CH

_mesh = plsc.VectorSubcoreMesh(core_axis_name="c", subcore_axis_name="s")
_sc_params = pltpu.CompilerParams(use_tc_tiling_on_sc=False)


@functools.partial(
    pl.kernel,
    out_type=jax.ShapeDtypeStruct((_BATCH, _DIM), jnp.float32),
    mesh=_mesh,
    compiler_params=_sc_params,
    scratch_types=[
        pltpu.VMEM((_NCH, _CH), jnp.int32),
        pltpu.VMEM((_BPW, _DIM), jnp.float32),
        pltpu.VMEM((_BPW, _DIM), jnp.float32),
        pltpu.SemaphoreType.DMA,
    ],
)
def _sc_update(idx_hbm, data_hbm, bank_ref, avgs_hbm,
               idx_v, rows_v, data_v, sem):
    w = lax.axis_index("s") * _NC + lax.axis_index("c")
    base = w * _BPW

    pltpu.sync_copy(idx_hbm.at[pl.ds(w * _NCH, _NCH)], idx_v)

    for j in range(_NCH):
        pltpu.async_copy(bank_ref.at[idx_v.at[j]],
                         rows_v.at[pl.ds(j * _CH, _CH)], sem)
    for j in range(_NCH):
        pltpu.make_async_copy(bank_ref.at[idx_v.at[j]],
                              rows_v.at[pl.ds(j * _CH, _CH)], sem).wait()

    pltpu.sync_copy(rows_v, avgs_hbm.at[pl.ds(base, _BPW)])
    pltpu.sync_copy(data_hbm.at[pl.ds(base, _BPW)], data_v)

    def body(i, carry):
        for k in range(_DIM // 16):
            sl = pl.ds(k * 16, 16)
            rows_v[i, sl] = rows_v[i, sl] * _MOM + data_v[i, sl] * (1.0 - _MOM)
        return carry

    lax.fori_loop(0, _BPW, body, 0)

    for j in range(_NCH):
        pltpu.async_copy(rows_v.at[pl.ds(j * _CH, _CH)],
                         bank_ref.at[idx_v.at[j]], sem)
    for j in range(_NCH):
        pltpu.make_async_copy(rows_v.at[pl.ds(j * _CH, _CH)],
                              bank_ref.at[idx_v.at[j]], sem).wait()


def _avgs_t_body(x_ref, o_ref):
    o_ref[...] = x_ref[...].T


def _avgs_transpose(avgs_rm):
    return pl.pallas_call(
        _avgs_t_body,
        grid=(16,),
        in_specs=[pl.BlockSpec((1024, _DIM), lambda i: (i, 0))],
        out_specs=pl.BlockSpec((_DIM, 1024), lambda i: (0, i)),
        out_shape=jax.ShapeDtypeStruct((_DIM, _BATCH), jnp.float32),
    )(avgs_rm)


def kernel(idx, data, memory):
    idx2d = idx.astype(jnp.int32).reshape(_NW * _NCH, _CH)
    bank_ref = jax.new_ref(memory)
    avgs_rm = _sc_update(idx2d, data, bank_ref)
    return _avgs_transpose(avgs_rm).T, bank_ref[...]


# speedup vs baseline: 23.2097x; 1.6565x over previous
import functools

import jax
import jax.numpy as jnp
from jax import lax
from jax.experimental import pallas as pl
from jax.experimental.pallas import tpu as pltpu
from jax.experimental.pallas import tpu_sc as plsc

_BANK = 1000001
_DIM = 64
_BATCH = 16384
_MOM = 0.9

_NC, _NS = 2, 16
_NW = _NC * _NS
_BPW = _BATCH // _NW
_RND = 256
_G = 16
_NG = _RND // _G

_mesh = plsc.VectorSubcoreMesh(core_axis_name="c", subcore_axis_name="s")
_sc_params = pltpu.CompilerParams(use_tc_tiling_on_sc=True)


@functools.partial(
    pl.kernel,
    out_type=jax.ShapeDtypeStruct((_BATCH, _DIM), jnp.float32),
    mesh=_mesh,
    compiler_params=_sc_params,
    scratch_types=[
        pltpu.VMEM((_BPW,), jnp.int32),
        pltpu.VMEM((_RND, _DIM), jnp.float32),
        pltpu.VMEM((_RND, _DIM), jnp.float32),
        pltpu.SemaphoreType.DMA,
    ],
)
def _sc_update(idx_hbm, data_hbm, bank_ref, avgs_hbm,
               idx_v, rows_v, data_v, sem):
    w = lax.axis_index("s") * _NC + lax.axis_index("c")
    base = w * _BPW

    pltpu.sync_copy(idx_hbm.at[pl.ds(base, _BPW)], idx_v)

    def drain_rows(g):
        pltpu.make_async_copy(bank_ref.at[pl.ds(0, _G)],
                              rows_v.at[pl.ds(g * _G, _G)], sem).wait()

    def drain_scatter(g):
        pltpu.make_async_copy(rows_v.at[pl.ds(g * _G, _G)],
                              bank_ref.at[pl.ds(0, _G)], sem).wait()

    for h in range(_BPW // _RND):
        hb = h * _RND

        def gbody(g, carry):
            v = idx_v[pl.ds(hb + g * _G, _G)]
            for j in range(_G):
                pltpu.async_copy(bank_ref.at[pl.ds(v[j], 1)],
                                 rows_v.at[pl.ds(g * _G + j, 1)], sem)

            @pl.when(g > 0)
            def _():
                drain_rows(g - 1)
            return carry

        lax.fori_loop(0, _NG, gbody, 0)
        drain_rows(_NG - 1)

        pltpu.sync_copy(rows_v, avgs_hbm.at[pl.ds(base + hb, _RND)])
        pltpu.sync_copy(data_hbm.at[pl.ds(base + hb, _RND)], data_v)

        def cbody(i, carry):
            for k in range(_DIM // 16):
                sl = pl.ds(k * 16, 16)
                rows_v[i, sl] = (rows_v[i, sl] * _MOM
                                 + data_v[i, sl] * (1.0 - _MOM))
            return carry

        lax.fori_loop(0, _RND, cbody, 0)

        def sbody(g, carry):
            v = idx_v[pl.ds(hb + g * _G, _G)]
            for j in range(_G):
                pltpu.async_copy(rows_v.at[pl.ds(g * _G + j, 1)],
                                 bank_ref.at[pl.ds(v[j], 1)], sem)

            @pl.when(g > 0)
            def _():
                drain_scatter(g - 1)
            return carry

        lax.fori_loop(0, _NG, sbody, 0)
        drain_scatter(_NG - 1)


def _avgs_t_body(x_ref, o_ref):
    o_ref[...] = x_ref[...].T


def _avgs_transpose(avgs_rm):
    return pl.pallas_call(
        _avgs_t_body,
        grid=(16,),
        in_specs=[pl.BlockSpec((1024, _DIM), lambda i: (i, 0))],
        out_specs=pl.BlockSpec((_DIM, 1024), lambda i: (0, i)),
        out_shape=jax.ShapeDtypeStruct((_DIM, _BATCH), jnp.float32),
    )(avgs_rm)


def kernel(idx, data, memory):
    idx = idx.astype(jnp.int32)
    bank_ref = jax.new_ref(memory)
    avgs_rm = _sc_update(idx, data, bank_ref)
    return _avgs_transpose(avgs_rm).T, bank_ref[...]


# speedup vs baseline: 23.2308x; 1.0009x over previous
import functools

import jax
import jax.numpy as jnp
from jax import lax
from jax.experimental import pallas as pl
from jax.experimental.pallas import tpu as pltpu
from jax.experimental.pallas import tpu_sc as plsc

_BANK = 1000001
_DIM = 64
_BATCH = 16384
_MOM = 0.9

_NC, _NS = 2, 16
_NW = _NC * _NS
_BPW = _BATCH // _NW
_RND = 256
_G = 16
_NG = _RND // _G

_mesh = plsc.VectorSubcoreMesh(core_axis_name="c", subcore_axis_name="s")
_sc_params = pltpu.CompilerParams(use_tc_tiling_on_sc=True)


@functools.partial(
    pl.kernel,
    out_type=jax.ShapeDtypeStruct((_BATCH, _DIM), jnp.float32),
    mesh=_mesh,
    compiler_params=_sc_params,
    scratch_types=[
        pltpu.VMEM((_BPW,), jnp.int32),
        pltpu.VMEM((_RND, _DIM), jnp.float32),
        pltpu.VMEM((_RND, _DIM), jnp.float32),
        pltpu.VMEM((_RND, _DIM), jnp.float32),
        pltpu.SemaphoreType.DMA,
    ],
)
def _sc_update(idx_hbm, data_hbm, bank_ref, avgs_hbm,
               idx_v, rows0_v, rows1_v, data_v, sem):
    w = lax.axis_index("s") * _NC + lax.axis_index("c")
    base = w * _BPW

    pltpu.sync_copy(idx_hbm.at[pl.ds(base, _BPW)], idx_v)

    def gather_into(rows_v, hb):
        def drain(g):
            pltpu.make_async_copy(bank_ref.at[pl.ds(0, _G)],
                                  rows_v.at[pl.ds(g * _G, _G)], sem).wait()

        def gbody(g, carry):
            v = idx_v[pl.ds(hb + g * _G, _G)]
            for j in range(_G):
                pltpu.async_copy(bank_ref.at[pl.ds(v[j], 1)],
                                 rows_v.at[pl.ds(g * _G + j, 1)], sem)

            @pl.when(g > 0)
            def _():
                drain(g - 1)
            return carry

        lax.fori_loop(0, _NG, gbody, 0)
        drain(_NG - 1)

    gather_into(rows0_v, 0)
    gather_into(rows1_v, _RND)

    pltpu.sync_copy(rows0_v, avgs_hbm.at[pl.ds(base, _RND)])
    pltpu.sync_copy(rows1_v, avgs_hbm.at[pl.ds(base + _RND, _RND)])

    for h, rows_v in enumerate((rows0_v, rows1_v)):
        pltpu.sync_copy(data_hbm.at[pl.ds(base + h * _RND, _RND)], data_v)

        def cbody(i, carry, rows_v=rows_v):
            for k in range(_DIM // 16):
                sl = pl.ds(k * 16, 16)
                rows_v[i, sl] = (rows_v[i, sl] * _MOM
                                 + data_v[i, sl] * (1.0 - _MOM))
            return carry

        lax.fori_loop(0, _RND, cbody, 0)

    for h, rows_v in enumerate((rows0_v, rows1_v)):
        def drain_s(g, rows_v=rows_v):
            pltpu.make_async_copy(rows_v.at[pl.ds(g * _G, _G)],
                                  bank_ref.at[pl.ds(0, _G)], sem).wait()

        def sbody(g, carry, rows_v=rows_v, hb=h * _RND):
            v = idx_v[pl.ds(hb + g * _G, _G)]
            for j in range(_G):
                pltpu.async_copy(rows_v.at[pl.ds(g * _G + j, 1)],
                                 bank_ref.at[pl.ds(v[j], 1)], sem)

            @pl.when(g > 0)
            def _():
                drain_s(g - 1)
            return carry

        lax.fori_loop(0, _NG, sbody, 0)
        drain_s(_NG - 1)


def _avgs_t_body(x_ref, o_ref):
    o_ref[...] = x_ref[...].T


def _avgs_transpose(avgs_rm):
    return pl.pallas_call(
        _avgs_t_body,
        grid=(16,),
        in_specs=[pl.BlockSpec((1024, _DIM), lambda i: (i, 0))],
        out_specs=pl.BlockSpec((_DIM, 1024), lambda i: (0, i)),
        out_shape=jax.ShapeDtypeStruct((_DIM, _BATCH), jnp.float32),
    )(avgs_rm)


def kernel(idx, data, memory):
    idx = idx.astype(jnp.int32)
    bank_ref = jax.new_ref(memory)
    avgs_rm = _sc_update(idx, data, bank_ref)
    return _avgs_transpose(avgs_rm).T, bank_ref[...]
